# Initial kernel scaffold; baseline (speedup 1.0000x reference)
import jax
import jax.numpy as jnp
from jax.experimental import pallas as pl


def kernel(x, edge_index, k_values, weighting, W, b):
    raise NotImplementedError("write your pallas kernel here")


# speedup vs baseline: 3.2185x; 3.2185x over previous
import functools

import jax
import jax.numpy as jnp
from jax import lax
from jax.experimental import pallas as pl
from jax.experimental.pallas import tpu as pltpu
from jax.experimental.pallas import tpu_sc as plsc

N = 10000
D = 128
EN = 320000
NC, NS, L = 2, 16, 16
NW = NC * NS
NP = 10240
RPT = NP // NW
NPH = NP + 128
ZR = NPH // NS
HALF = NP // NC
AGGR = HALF + 128
ARPT = AGGR // NS
EPT = 10240
EP = EPT * NW
ECH = 128
EG = 2048
NCG = EPT // EG
CPG = EG // ECH
DCH = 64
NDC = RPT // DCH
EPS = EP // NS
NCGS = EPS // EG

_mesh = plsc.VectorSubcoreMesh(
    core_axis_name="c", subcore_axis_name="s", num_cores=NC, num_subcores=NS)


@functools.partial(
    pl.kernel,
    out_type=[
        jax.ShapeDtypeStruct((NC * NPH, D), jnp.float32),
    ],
    mesh=_mesh,
    scratch_types=[
        pltpu.VMEM((EG,), jnp.int32),
        pltpu.VMEM((ECH,), jnp.int32),
        pltpu.VMEM((ECH, D), jnp.float32),
        pltpu.VMEM_SHARED((NPH, D), jnp.float32),
    ],
)
def _deg_kernel(colp, ones_in, zeros_in, partials, colbuf, idxbuf, ones,
                deg_sh):
    c = lax.axis_index("c")
    s = lax.axis_index("s")
    wid = c * NS + s

    pltpu.sync_copy(ones_in, ones)
    pltpu.sync_copy(zeros_in.at[pl.ds(0, ZR)], deg_sh.at[pl.ds(s * ZR, ZR)])
    plsc.subcore_barrier()

    def group(g, _):
        gb = wid * EPT + g * EG
        pltpu.sync_copy(colp.at[pl.ds(gb, EG)], colbuf)

        def chunk(k, _):
            def cpy(i, _):
                idxbuf[pl.ds(i * L, L)] = colbuf[pl.ds(k * ECH + i * L, L)]
                return 0
            lax.fori_loop(0, ECH // L, cpy, 0)
            pltpu.sync_copy(ones, deg_sh.at[idxbuf], add=True)
            return 0
        lax.fori_loop(0, CPG, chunk, 0)
        return 0
    lax.fori_loop(0, NCG, group, 0)
    plsc.subcore_barrier()

    pltpu.sync_copy(deg_sh.at[pl.ds(s * ZR, ZR)],
                    partials.at[pl.ds(c * NPH + s * ZR, ZR)])


@functools.partial(
    pl.kernel,
    out_type=[
        jax.ShapeDtypeStruct((NP, L), jnp.float32),
        jax.ShapeDtypeStruct((NP, D), jnp.float32),
        jax.ShapeDtypeStruct((L,), jnp.float32),
    ],
    mesh=_mesh,
    scratch_types=[
        pltpu.VMEM((DCH, D), jnp.float32),
        pltpu.VMEM((DCH, D), jnp.float32),
        pltpu.VMEM((DCH, L), jnp.float32),
        pltpu.VMEM((DCH, D), jnp.float32),
        pltpu.VMEM((2, L), jnp.float32),
        pltpu.VMEM((L,), jnp.float32),
    ],
)
def _dis_kernel(partials, xp, kw, disb, g0, coef, pbufa, pbufb, dsb, xbuf,
                kbuf, cbuf):
    c = lax.axis_index("c")
    s = lax.axis_index("s")
    wid = c * NS + s
    base = wid * RPT
    iot = lax.iota(jnp.int32, L)

    def dchunk(ci, _):
        o = base + ci * DCH
        pltpu.sync_copy(partials.at[pl.ds(o, DCH)], pbufa)
        pltpu.sync_copy(partials.at[pl.ds(NPH + o, DCH)], pbufb)
        pltpu.sync_copy(xp.at[pl.ds(o, DCH)], xbuf)

        def rbody(gi, _):
            for i in range(L):
                r = gi * L + i
                deg = pbufa[r, pl.ds(0, L)] + pbufb[r, pl.ds(0, L)]
                deg = jnp.where(deg == 0.0, 1.0, deg)
                y = 1.0 / deg
                for _ in range(24):
                    y = y * (1.5 - 0.5 * deg * y * y)
                dsb[r, :] = y
                for j in range(D // L):
                    sl = pl.ds(j * L, L)
                    xbuf[r, sl] = xbuf[r, sl] * y
            return 0
        lax.fori_loop(0, DCH // L, rbody, 0)
        pltpu.sync_copy(dsb, disb.at[pl.ds(o, DCH)])
        pltpu.sync_copy(xbuf, g0.at[pl.ds(o, DCH)])
        return 0
    lax.fori_loop(0, NDC, dchunk, 0)

    @pl.when(wid == 0)
    def _():
        pltpu.sync_copy(kw, kbuf)
        kv = kbuf[0, :]
        e2k = jnp.exp(2.0 * kv)
        th = 1.0 - 2.0 / (e2k + 1.0)
        csp = 1.0 / (1.0 + jnp.exp(-kbuf[1, :]))
        cv = jnp.where(iot < 3, csp * th,
                       jnp.where(iot == 3, 1.0 - csp, 0.0))
        cbuf[...] = cv
        pltpu.sync_copy(cbuf, coef)


@functools.partial(
    pl.kernel,
    out_type=[
        jax.ShapeDtypeStruct((NP, D), jnp.float32),
        jax.ShapeDtypeStruct((NP, D), jnp.float32),
    ],
    mesh=_mesh,
    scratch_types=[
        pltpu.VMEM((EG,), jnp.int32),
        pltpu.VMEM((EG,), jnp.int32),
        pltpu.VMEM((ECH,), jnp.int32),
        pltpu.VMEM((ECH,), jnp.int32),
        pltpu.VMEM((ECH, D), jnp.float32),
        pltpu.VMEM((DCH, D), jnp.float32),
        pltpu.VMEM((DCH, D), jnp.float32),
        pltpu.VMEM((DCH, L), jnp.float32),
        pltpu.VMEM_SHARED((AGGR, D), jnp.float32),
        pltpu.SemaphoreType.DMA,
    ],
)
def _layer(rowp, colp, g_in, h_in, disb, zeros_in, h_out, g_out, rgbuf, cgbuf,
           rbuf, idxbuf, gbuf, aggbuf, hbuf, dbuf, agg_sh, sem):
    c = lax.axis_index("c")
    s = lax.axis_index("s")
    wid = c * NS + s
    cbase = c * HALF

    pltpu.sync_copy(zeros_in.at[pl.ds(0, ARPT)],
                    agg_sh.at[pl.ds(s * ARPT, ARPT)])
    plsc.subcore_barrier()

    def group(g, _):
        gb = s * EPS + g * EG
        pltpu.sync_copy(rowp.at[pl.ds(gb, EG)], rgbuf)
        pltpu.sync_copy(colp.at[pl.ds(gb, EG)], cgbuf)

        def chunk(k, _):
            def cpy(i, _):
                rbuf[pl.ds(i * L, L)] = rgbuf[pl.ds(k * ECH + i * L, L)]
                cv = cgbuf[pl.ds(k * ECH + i * L, L)]
                lc = cv - cbase
                idxbuf[pl.ds(i * L, L)] = jnp.where(
                    (lc >= 0) & (lc < HALF), lc, HALF)
                return 0
            lax.fori_loop(0, ECH // L, cpy, 0)
            pltpu.async_copy(g_in.at[rbuf], gbuf, sem).wait()
            pltpu.sync_copy(gbuf, agg_sh.at[idxbuf], add=True)
            return 0
        lax.fori_loop(0, CPG, chunk, 0)
        return 0
    lax.fori_loop(0, NCGS, group, 0)
    plsc.subcore_barrier()

    def dchunk(ci, _):
        o = ci * DCH
        lb = s * RPT + o
        gb = c * HALF + s * RPT + o
        pltpu.sync_copy(agg_sh.at[pl.ds(lb, DCH)], aggbuf)
        pltpu.sync_copy(h_in.at[pl.ds(gb, DCH)], hbuf)
        pltpu.sync_copy(disb.at[pl.ds(gb, DCH)], dbuf)

        def dense(gi, _):
            for i in range(L):
                r = gi * L + i
                sp = dbuf[r, :]
                for j in range(D // L):
                    sl = pl.ds(j * L, L)
                    hnew = hbuf[r, sl] - sp * aggbuf[r, sl]
                    hbuf[r, sl] = hnew
                    aggbuf[r, sl] = sp * hnew
            return 0
        lax.fori_loop(0, DCH // L, dense, 0)
        pltpu.sync_copy(hbuf, h_out.at[pl.ds(gb, DCH)])
        pltpu.sync_copy(aggbuf, g_out.at[pl.ds(gb, DCH)])
        return 0
    lax.fori_loop(0, NDC, dchunk, 0)


_BM = 256


def _final_body(coef_ref, x_ref, h1_ref, h2_ref, h3_ref, w_ref, b_ref, o_ref):
    s0 = coef_ref[0]
    s1 = coef_ref[1]
    s2 = coef_ref[2]
    s3 = coef_ref[3]
    mixed = (s3 * x_ref[...] + s0 * h1_ref[...] + s1 * h2_ref[...]
             + s2 * h3_ref[...])
    acc = lax.dot_general(mixed, w_ref[...], (((1,), (1,)), ((), ())),
                          preferred_element_type=jnp.float32)
    o_ref[...] = jnp.maximum(acc + b_ref[...], 0.0)


_final_call = pl.pallas_call(
    _final_body,
    grid=(NP // _BM,),
    in_specs=[
        pl.BlockSpec(memory_space=pltpu.SMEM),
        pl.BlockSpec((_BM, D), lambda i: (i, 0)),
        pl.BlockSpec((_BM, D), lambda i: (i, 0)),
        pl.BlockSpec((_BM, D), lambda i: (i, 0)),
        pl.BlockSpec((_BM, D), lambda i: (i, 0)),
        pl.BlockSpec((D, D), lambda i: (0, 0)),
        pl.BlockSpec((1, D), lambda i: (0, 0)),
    ],
    out_specs=pl.BlockSpec((_BM, D), lambda i: (i, 0)),
    out_shape=jax.ShapeDtypeStruct((NP, D), jnp.float32),
)


def kernel(x, edge_index, k_values, weighting, W, b):
    x = x.astype(jnp.float32)
    ei = edge_index.astype(jnp.int32)
    row, col = ei[0], ei[1]
    rowp = jnp.concatenate([row, jnp.zeros((EP - EN,), jnp.int32)])
    colp = jnp.concatenate([col, jnp.full((EP - EN,), NP, jnp.int32)])
    xp = jnp.concatenate([x, jnp.zeros((NP - N, D), jnp.float32)])
    kw0 = jnp.concatenate([k_values.reshape(3).astype(jnp.float32),
                           weighting.reshape(1).astype(jnp.float32),
                           jnp.zeros((L - 4,), jnp.float32)])
    kw1 = jnp.broadcast_to(weighting.reshape(1).astype(jnp.float32), (L,))
    kw = jnp.stack([kw0, kw1])
    ones_in = jnp.ones((ECH, D), jnp.float32)
    zeros_in = jnp.zeros((ZR, D), jnp.float32)

    (partials,) = _deg_kernel(colp, ones_in, zeros_in)
    disb, g0, coef = _dis_kernel(partials, xp, kw)

    def _body(carry, _):
        h, g = carry
        h_new, g_new = _layer(rowp, colp, g, h, disb, zeros_in)
        return (h_new, g_new), h_new

    _, hs = lax.scan(_body, (xp, g0), None, length=3)
    out = _final_call(coef, xp, hs[0], hs[1], hs[2], W, b.reshape(1, D))
    return out[:N]


# speedup vs baseline: 3.4028x; 1.0573x over previous
import functools

import jax
import jax.numpy as jnp
from jax import lax
from jax.experimental import pallas as pl
from jax.experimental.pallas import tpu as pltpu
from jax.experimental.pallas import tpu_sc as plsc

N = 10000
D = 128
EN = 320000
NC, NS, L = 2, 16, 16
NW = NC * NS
NP = 10240
RPT = NP // NW
NPH = NP + 128
ZR = NPH // NS
HALF = NP // NC
AGGR = HALF + 128
ARPT = AGGR // NS
EPT = 10240
EP = EPT * NW
ECH = 128
EG = 2048
NCG = EPT // EG
CPG = EG // ECH
DCH = 64
NDC = RPT // DCH
EPS = EP // NS
NCGS = EPS // EG

_mesh = plsc.VectorSubcoreMesh(
    core_axis_name="c", subcore_axis_name="s", num_cores=NC, num_subcores=NS)


@functools.partial(
    pl.kernel,
    out_type=[
        jax.ShapeDtypeStruct((NC * NPH, D), jnp.float32),
    ],
    mesh=_mesh,
    scratch_types=[
        pltpu.VMEM((EG,), jnp.int32),
        pltpu.VMEM((ECH,), jnp.int32),
        pltpu.VMEM((ECH, D), jnp.float32),
        pltpu.VMEM_SHARED((NPH, D), jnp.float32),
    ],
)
def _deg_kernel(colp, ones_in, zeros_in, partials, colbuf, idxbuf, ones,
                deg_sh):
    c = lax.axis_index("c")
    s = lax.axis_index("s")
    wid = c * NS + s

    pltpu.sync_copy(ones_in, ones)
    pltpu.sync_copy(zeros_in.at[pl.ds(0, ZR)], deg_sh.at[pl.ds(s * ZR, ZR)])
    plsc.subcore_barrier()

    def group(g, _):
        gb = wid * EPT + g * EG
        pltpu.sync_copy(colp.at[pl.ds(gb, EG)], colbuf)

        def chunk(k, _):
            def cpy(i, _):
                idxbuf[pl.ds(i * L, L)] = colbuf[pl.ds(k * ECH + i * L, L)]
                return 0
            lax.fori_loop(0, ECH // L, cpy, 0)
            pltpu.sync_copy(ones, deg_sh.at[idxbuf], add=True)
            return 0
        lax.fori_loop(0, CPG, chunk, 0)
        return 0
    lax.fori_loop(0, NCG, group, 0)
    plsc.subcore_barrier()

    pltpu.sync_copy(deg_sh.at[pl.ds(s * ZR, ZR)],
                    partials.at[pl.ds(c * NPH + s * ZR, ZR)])


@functools.partial(
    pl.kernel,
    out_type=[
        jax.ShapeDtypeStruct((NP, L), jnp.float32),
        jax.ShapeDtypeStruct((NP, D), jnp.float32),
        jax.ShapeDtypeStruct((L,), jnp.float32),
    ],
    mesh=_mesh,
    scratch_types=[
        pltpu.VMEM((DCH, D), jnp.float32),
        pltpu.VMEM((DCH, D), jnp.float32),
        pltpu.VMEM((DCH, L), jnp.float32),
        pltpu.VMEM((DCH, D), jnp.float32),
        pltpu.VMEM((2, L), jnp.float32),
        pltpu.VMEM((L,), jnp.float32),
    ],
)
def _dis_kernel(partials, xp, kw, disb, g0, coef, pbufa, pbufb, dsb, xbuf,
                kbuf, cbuf):
    c = lax.axis_index("c")
    s = lax.axis_index("s")
    wid = c * NS + s
    base = wid * RPT
    iot = lax.iota(jnp.int32, L)

    def dchunk(ci, _):
        o = base + ci * DCH
        pltpu.sync_copy(partials.at[pl.ds(o, DCH)], pbufa)
        pltpu.sync_copy(partials.at[pl.ds(NPH + o, DCH)], pbufb)
        pltpu.sync_copy(xp.at[pl.ds(o, DCH)], xbuf)

        def rbody(gi, _):
            for i in range(L):
                r = gi * L + i
                deg = pbufa[r, pl.ds(0, L)] + pbufb[r, pl.ds(0, L)]
                deg = jnp.where(deg == 0.0, 1.0, deg)
                y = 1.0 / deg
                for _ in range(24):
                    y = y * (1.5 - 0.5 * deg * y * y)
                dsb[r, :] = y
                for j in range(D // L):
                    sl = pl.ds(j * L, L)
                    xbuf[r, sl] = xbuf[r, sl] * y
            return 0
        lax.fori_loop(0, DCH // L, rbody, 0)
        pltpu.sync_copy(dsb, disb.at[pl.ds(o, DCH)])
        pltpu.sync_copy(xbuf, g0.at[pl.ds(o, DCH)])
        return 0
    lax.fori_loop(0, NDC, dchunk, 0)

    @pl.when(wid == 0)
    def _():
        pltpu.sync_copy(kw, kbuf)
        kv = kbuf[0, :]
        e2k = jnp.exp(2.0 * kv)
        th = 1.0 - 2.0 / (e2k + 1.0)
        csp = 1.0 / (1.0 + jnp.exp(-kbuf[1, :]))
        cv = jnp.where(iot < 3, csp * th,
                       jnp.where(iot == 3, 1.0 - csp, 0.0))
        cbuf[...] = cv
        pltpu.sync_copy(cbuf, coef)


@functools.partial(
    pl.kernel,
    out_type=[
        jax.ShapeDtypeStruct((NP, D), jnp.float32),
        jax.ShapeDtypeStruct((NP, D), jnp.float32),
    ],
    mesh=_mesh,
    scratch_types=[
        pltpu.VMEM((EG,), jnp.int32),
        pltpu.VMEM((EG,), jnp.int32),
        pltpu.VMEM((ECH,), jnp.int32),
        pltpu.VMEM((ECH,), jnp.int32),
        pltpu.VMEM((ECH,), jnp.int32),
        pltpu.VMEM((ECH,), jnp.int32),
        pltpu.VMEM((ECH, D), jnp.float32),
        pltpu.VMEM((ECH, D), jnp.float32),
        pltpu.VMEM((DCH, D), jnp.float32),
        pltpu.VMEM((DCH, D), jnp.float32),
        pltpu.VMEM((DCH, L), jnp.float32),
        pltpu.VMEM_SHARED((AGGR, D), jnp.float32),
        pltpu.SemaphoreType.DMA,
        pltpu.SemaphoreType.DMA,
    ],
)
def _layer(rowp, colp, g_in, h_in, disb, zeros_in, h_out, g_out, rgbuf, cgbuf,
           rbuf0, rbuf1, idxbuf0, idxbuf1, gbuf0, gbuf1, aggbuf, hbuf, dbuf,
           agg_sh, sem0, sem1):
    c = lax.axis_index("c")
    s = lax.axis_index("s")
    cbase = c * HALF
    rbufs = (rbuf0, rbuf1)
    idxbufs = (idxbuf0, idxbuf1)
    gbufs = (gbuf0, gbuf1)
    sems = (sem0, sem1)

    pltpu.sync_copy(zeros_in.at[pl.ds(0, ARPT)],
                    agg_sh.at[pl.ds(s * ARPT, ARPT)])
    plsc.subcore_barrier()

    def build_idx(k, rb, ib):
        def cpy(i, _):
            rb[pl.ds(i * L, L)] = rgbuf[pl.ds(k * ECH + i * L, L)]
            cv = cgbuf[pl.ds(k * ECH + i * L, L)]
            lc = cv - cbase
            ib[pl.ds(i * L, L)] = jnp.where(
                (lc >= 0) & (lc < HALF), lc, HALF)
            return 0
        lax.fori_loop(0, ECH // L, cpy, 0)

    def group(g, _):
        gb = s * EPS + g * EG
        pltpu.sync_copy(rowp.at[pl.ds(gb, EG)], rgbuf)
        pltpu.sync_copy(colp.at[pl.ds(gb, EG)], cgbuf)

        descs = [None, None]
        for k in range(CPG):
            b = k % 2
            build_idx(k, rbufs[b], idxbufs[b])
            descs[b] = pltpu.async_copy(g_in.at[rbufs[b]], gbufs[b], sems[b])
            if k > 0:
                descs[1 - b].wait()
                pltpu.sync_copy(gbufs[1 - b], agg_sh.at[idxbufs[1 - b]],
                                add=True)
        descs[1].wait()
        pltpu.sync_copy(gbufs[1], agg_sh.at[idxbufs[1]], add=True)
        return 0
    lax.fori_loop(0, NCGS, group, 0)
    plsc.subcore_barrier()

    def dchunk(ci, _):
        o = ci * DCH
        lb = s * RPT + o
        gb = c * HALF + s * RPT + o
        pltpu.sync_copy(agg_sh.at[pl.ds(lb, DCH)], aggbuf)
        pltpu.sync_copy(h_in.at[pl.ds(gb, DCH)], hbuf)
        pltpu.sync_copy(disb.at[pl.ds(gb, DCH)], dbuf)

        def dense(gi, _):
            for i in range(L):
                r = gi * L + i
                sp = dbuf[r, :]
                for j in range(D // L):
                    sl = pl.ds(j * L, L)
                    hnew = hbuf[r, sl] - sp * aggbuf[r, sl]
                    hbuf[r, sl] = hnew
                    aggbuf[r, sl] = sp * hnew
            return 0
        lax.fori_loop(0, DCH // L, dense, 0)
        pltpu.sync_copy(hbuf, h_out.at[pl.ds(gb, DCH)])
        pltpu.sync_copy(aggbuf, g_out.at[pl.ds(gb, DCH)])
        return 0
    lax.fori_loop(0, NDC, dchunk, 0)


_BM = 256


def _final_body(coef_ref, x_ref, h1_ref, h2_ref, h3_ref, w_ref, b_ref, o_ref):
    s0 = coef_ref[0]
    s1 = coef_ref[1]
    s2 = coef_ref[2]
    s3 = coef_ref[3]
    mixed = (s3 * x_ref[...] + s0 * h1_ref[...] + s1 * h2_ref[...]
             + s2 * h3_ref[...])
    acc = lax.dot_general(mixed, w_ref[...], (((1,), (1,)), ((), ())),
                          preferred_element_type=jnp.float32)
    o_ref[...] = jnp.maximum(acc + b_ref[...], 0.0)


_final_call = pl.pallas_call(
    _final_body,
    grid=(NP // _BM,),
    in_specs=[
        pl.BlockSpec(memory_space=pltpu.SMEM),
        pl.BlockSpec((_BM, D), lambda i: (i, 0)),
        pl.BlockSpec((_BM, D), lambda i: (i, 0)),
        pl.BlockSpec((_BM, D), lambda i: (i, 0)),
        pl.BlockSpec((_BM, D), lambda i: (i, 0)),
        pl.BlockSpec((D, D), lambda i: (0, 0)),
        pl.BlockSpec((1, D), lambda i: (0, 0)),
    ],
    out_specs=pl.BlockSpec((_BM, D), lambda i: (i, 0)),
    out_shape=jax.ShapeDtypeStruct((NP, D), jnp.float32),
)


def kernel(x, edge_index, k_values, weighting, W, b):
    x = x.astype(jnp.float32)
    ei = edge_index.astype(jnp.int32)
    row, col = ei[0], ei[1]
    rowp = jnp.concatenate([row, jnp.zeros((EP - EN,), jnp.int32)])
    colp = jnp.concatenate([col, jnp.full((EP - EN,), NP, jnp.int32)])
    xp = jnp.concatenate([x, jnp.zeros((NP - N, D), jnp.float32)])
    kw0 = jnp.concatenate([k_values.reshape(3).astype(jnp.float32),
                           weighting.reshape(1).astype(jnp.float32),
                           jnp.zeros((L - 4,), jnp.float32)])
    kw1 = jnp.broadcast_to(weighting.reshape(1).astype(jnp.float32), (L,))
    kw = jnp.stack([kw0, kw1])
    ones_in = jnp.ones((ECH, D), jnp.float32)
    zeros_in = jnp.zeros((ZR, D), jnp.float32)

    (partials,) = _deg_kernel(colp, ones_in, zeros_in)
    disb, g0, coef = _dis_kernel(partials, xp, kw)

    def _body(carry, _):
        h, g = carry
        h_new, g_new = _layer(rowp, colp, g, h, disb, zeros_in)
        return (h_new, g_new), h_new

    _, hs = lax.scan(_body, (xp, g0), None, length=3)
    out = _final_call(coef, xp, hs[0], hs[1], hs[2], W, b.reshape(1, D))
    return out[:N]


# speedup vs baseline: 5.4106x; 1.5900x over previous
import functools

import jax
import jax.numpy as jnp
from jax import lax
from jax.experimental import pallas as pl
from jax.experimental.pallas import tpu as pltpu
from jax.experimental.pallas import tpu_sc as plsc

N = 10000
D = 128
EN = 320000
NC, NS, L = 2, 16, 16
NW = NC * NS
NP = 10240
RPT = NP // NW
NPH = NP + 128
ZR = NPH // NS
HALF = NP // NC
AGGR = HALF + 128
ARPT = AGGR // NS
EPT = 10240
EP = EPT * NW
ECH = 128
EG = 2048
NCG = EPT // EG
CPG = EG // ECH
DCH = 64
NDC = RPT // DCH
EPS = EP // NS
NCGS = EPS // EG

_mesh = plsc.VectorSubcoreMesh(
    core_axis_name="c", subcore_axis_name="s", num_cores=NC, num_subcores=NS)


@functools.partial(
    pl.kernel,
    out_type=[
        jax.ShapeDtypeStruct((NC * NPH, D), jnp.float32),
    ],
    mesh=_mesh,
    scratch_types=[
        pltpu.VMEM((EG,), jnp.int32),
        pltpu.VMEM((ECH,), jnp.int32),
        pltpu.VMEM((ECH, D), jnp.float32),
        pltpu.VMEM_SHARED((NPH, D), jnp.float32),
    ],
)
def _deg_kernel(colp, ones_in, zeros_in, partials, colbuf, idxbuf, ones,
                deg_sh):
    c = lax.axis_index("c")
    s = lax.axis_index("s")
    wid = c * NS + s

    pltpu.sync_copy(ones_in, ones)
    pltpu.sync_copy(zeros_in.at[pl.ds(0, ZR)], deg_sh.at[pl.ds(s * ZR, ZR)])
    plsc.subcore_barrier()

    def group(g, _):
        gb = wid * EPT + g * EG
        pltpu.sync_copy(colp.at[pl.ds(gb, EG)], colbuf)

        def chunk(k, _):
            def cpy(i, _):
                idxbuf[pl.ds(i * L, L)] = colbuf[pl.ds(k * ECH + i * L, L)]
                return 0
            lax.fori_loop(0, ECH // L, cpy, 0)
            pltpu.sync_copy(ones, deg_sh.at[idxbuf], add=True)
            return 0
        lax.fori_loop(0, CPG, chunk, 0)
        return 0
    lax.fori_loop(0, NCG, group, 0)
    plsc.subcore_barrier()

    pltpu.sync_copy(deg_sh.at[pl.ds(s * ZR, ZR)],
                    partials.at[pl.ds(c * NPH + s * ZR, ZR)])


@functools.partial(
    pl.kernel,
    out_type=[
        jax.ShapeDtypeStruct((NP, L), jnp.float32),
        jax.ShapeDtypeStruct((NP, D), jnp.float32),
        jax.ShapeDtypeStruct((L,), jnp.float32),
    ],
    mesh=_mesh,
    scratch_types=[
        pltpu.VMEM((DCH, D), jnp.float32),
        pltpu.VMEM((DCH, D), jnp.float32),
        pltpu.VMEM((DCH, L), jnp.float32),
        pltpu.VMEM((DCH, D), jnp.float32),
        pltpu.VMEM((2, L), jnp.float32),
        pltpu.VMEM((L,), jnp.float32),
    ],
)
def _dis_kernel(partials, xp, kw, disb, g0, coef, pbufa, pbufb, dsb, xbuf,
                kbuf, cbuf):
    c = lax.axis_index("c")
    s = lax.axis_index("s")
    wid = c * NS + s
    base = wid * RPT
    iot = lax.iota(jnp.int32, L)

    def dchunk(ci, _):
        o = base + ci * DCH
        pltpu.sync_copy(partials.at[pl.ds(o, DCH)], pbufa)
        pltpu.sync_copy(partials.at[pl.ds(NPH + o, DCH)], pbufb)
        pltpu.sync_copy(xp.at[pl.ds(o, DCH)], xbuf)

        def rbody(gi, _):
            for i in range(L):
                r = gi * L + i
                deg = pbufa[r, pl.ds(0, L)] + pbufb[r, pl.ds(0, L)]
                deg = jnp.where(deg == 0.0, 1.0, deg)
                y = 1.0 / deg
                for _ in range(24):
                    y = y * (1.5 - 0.5 * deg * y * y)
                dsb[r, :] = y
                for j in range(D // L):
                    sl = pl.ds(j * L, L)
                    xbuf[r, sl] = xbuf[r, sl] * y
            return 0
        lax.fori_loop(0, DCH // L, rbody, 0)
        pltpu.sync_copy(dsb, disb.at[pl.ds(o, DCH)])
        pltpu.sync_copy(xbuf, g0.at[pl.ds(o, DCH)])
        return 0
    lax.fori_loop(0, NDC, dchunk, 0)

    @pl.when(wid == 0)
    def _():
        pltpu.sync_copy(kw, kbuf)
        kv = kbuf[0, :]
        e2k = jnp.exp(2.0 * kv)
        th = 1.0 - 2.0 / (e2k + 1.0)
        csp = 1.0 / (1.0 + jnp.exp(-kbuf[1, :]))
        cv = jnp.where(iot < 3, csp * th,
                       jnp.where(iot == 3, 1.0 - csp, 0.0))
        cbuf[...] = cv
        pltpu.sync_copy(cbuf, coef)


@functools.partial(
    pl.kernel,
    out_type=[
        jax.ShapeDtypeStruct((NC * NP, D), jnp.float32),
    ],
    mesh=_mesh,
    scratch_types=[
        pltpu.VMEM((EG,), jnp.int32),
        pltpu.VMEM((EG,), jnp.int32),
        pltpu.VMEM((ECH,), jnp.int32),
        pltpu.VMEM((ECH,), jnp.int32),
        pltpu.VMEM((ECH,), jnp.int32),
        pltpu.VMEM((ECH,), jnp.int32),
        pltpu.VMEM((ECH, D), jnp.float32),
        pltpu.VMEM((ECH, D), jnp.float32),
        pltpu.VMEM_SHARED((NPH, D), jnp.float32),
        pltpu.SemaphoreType.DMA,
        pltpu.SemaphoreType.DMA,
    ],
)
def _scatter_layer(rowp, colp, g_in, zeros_in, aggout, rgbuf, cgbuf,
                   rbuf0, rbuf1, idxbuf0, idxbuf1, gbuf0, gbuf1, agg_sh,
                   sem0, sem1):
    c = lax.axis_index("c")
    s = lax.axis_index("s")
    wid = c * NS + s
    rbufs = (rbuf0, rbuf1)
    idxbufs = (idxbuf0, idxbuf1)
    gbufs = (gbuf0, gbuf1)
    sems = (sem0, sem1)

    pltpu.sync_copy(zeros_in, agg_sh.at[pl.ds(s * ZR, ZR)])
    plsc.subcore_barrier()

    def build_idx(k, rb, ib):
        def cpy(i, _):
            rb[pl.ds(i * L, L)] = rgbuf[pl.ds(k * ECH + i * L, L)]
            cv = cgbuf[pl.ds(k * ECH + i * L, L)]
            ib[pl.ds(i * L, L)] = jnp.minimum(cv, NP)
            return 0
        lax.fori_loop(0, ECH // L, cpy, 0)

    def group(g, _):
        gb = wid * EPT + g * EG
        pltpu.sync_copy(rowp.at[pl.ds(gb, EG)], rgbuf)
        pltpu.sync_copy(colp.at[pl.ds(gb, EG)], cgbuf)

        descs = [None, None]
        for k in range(CPG):
            b = k % 2
            build_idx(k, rbufs[b], idxbufs[b])
            descs[b] = pltpu.async_copy(g_in.at[rbufs[b]], gbufs[b], sems[b])
            if k > 0:
                descs[1 - b].wait()
                pltpu.sync_copy(gbufs[1 - b], agg_sh.at[idxbufs[1 - b]],
                                add=True)
        descs[1].wait()
        pltpu.sync_copy(gbufs[1], agg_sh.at[idxbufs[1]], add=True)
        return 0
    lax.fori_loop(0, NCG, group, 0)
    plsc.subcore_barrier()

    pltpu.sync_copy(agg_sh.at[pl.ds(s * (NP // NS), NP // NS)],
                    aggout.at[pl.ds(c * NP + s * (NP // NS), NP // NS)])


@functools.partial(
    pl.kernel,
    out_type=[
        jax.ShapeDtypeStruct((NP, D), jnp.float32),
        jax.ShapeDtypeStruct((NP, D), jnp.float32),
    ],
    mesh=_mesh,
    scratch_types=[
        pltpu.VMEM((DCH, D), jnp.float32),
        pltpu.VMEM((DCH, D), jnp.float32),
        pltpu.VMEM((DCH, D), jnp.float32),
        pltpu.VMEM((DCH, L), jnp.float32),
    ],
)
def _dense_layer(aggout, h_in, disb, h_out, g_out, bufa, bufb, hbuf, dbuf):
    c = lax.axis_index("c")
    s = lax.axis_index("s")
    wid = c * NS + s
    base = wid * RPT

    def dchunk(ci, _):
        gb = base + ci * DCH
        pltpu.sync_copy(aggout.at[pl.ds(gb, DCH)], bufa)
        pltpu.sync_copy(aggout.at[pl.ds(NP + gb, DCH)], bufb)
        pltpu.sync_copy(h_in.at[pl.ds(gb, DCH)], hbuf)
        pltpu.sync_copy(disb.at[pl.ds(gb, DCH)], dbuf)

        def dense(gi, _):
            for i in range(L):
                r = gi * L + i
                sp = dbuf[r, :]
                for j in range(D // L):
                    sl = pl.ds(j * L, L)
                    hnew = hbuf[r, sl] - sp * (bufa[r, sl] + bufb[r, sl])
                    hbuf[r, sl] = hnew
                    bufa[r, sl] = sp * hnew
            return 0
        lax.fori_loop(0, DCH // L, dense, 0)
        pltpu.sync_copy(hbuf, h_out.at[pl.ds(gb, DCH)])
        pltpu.sync_copy(bufa, g_out.at[pl.ds(gb, DCH)])
        return 0
    lax.fori_loop(0, NDC, dchunk, 0)


_BM = 256


def _final_body(coef_ref, x_ref, h1_ref, h2_ref, h3_ref, w_ref, b_ref, o_ref):
    s0 = coef_ref[0]
    s1 = coef_ref[1]
    s2 = coef_ref[2]
    s3 = coef_ref[3]
    mixed = (s3 * x_ref[...] + s0 * h1_ref[...] + s1 * h2_ref[...]
             + s2 * h3_ref[...])
    acc = lax.dot_general(mixed, w_ref[...], (((1,), (1,)), ((), ())),
                          preferred_element_type=jnp.float32)
    o_ref[...] = jnp.maximum(acc + b_ref[...], 0.0)


_final_call = pl.pallas_call(
    _final_body,
    grid=(NP // _BM,),
    in_specs=[
        pl.BlockSpec(memory_space=pltpu.SMEM),
        pl.BlockSpec((_BM, D), lambda i: (i, 0)),
        pl.BlockSpec((_BM, D), lambda i: (i, 0)),
        pl.BlockSpec((_BM, D), lambda i: (i, 0)),
        pl.BlockSpec((_BM, D), lambda i: (i, 0)),
        pl.BlockSpec((D, D), lambda i: (0, 0)),
        pl.BlockSpec((1, D), lambda i: (0, 0)),
    ],
    out_specs=pl.BlockSpec((_BM, D), lambda i: (i, 0)),
    out_shape=jax.ShapeDtypeStruct((NP, D), jnp.float32),
)


def kernel(x, edge_index, k_values, weighting, W, b):
    x = x.astype(jnp.float32)
    ei = edge_index.astype(jnp.int32)
    row, col = ei[0], ei[1]
    rowp = jnp.concatenate([row, jnp.zeros((EP - EN,), jnp.int32)])
    colp = jnp.concatenate([col, jnp.full((EP - EN,), NP, jnp.int32)])
    xp = jnp.concatenate([x, jnp.zeros((NP - N, D), jnp.float32)])
    kw0 = jnp.concatenate([k_values.reshape(3).astype(jnp.float32),
                           weighting.reshape(1).astype(jnp.float32),
                           jnp.zeros((L - 4,), jnp.float32)])
    kw1 = jnp.broadcast_to(weighting.reshape(1).astype(jnp.float32), (L,))
    kw = jnp.stack([kw0, kw1])
    ones_in = jnp.ones((ECH, D), jnp.float32)
    zeros_in = jnp.zeros((ZR, D), jnp.float32)

    (partials,) = _deg_kernel(colp, ones_in, zeros_in)
    disb, g0, coef = _dis_kernel(partials, xp, kw)

    def _body(carry, _):
        h, g = carry
        (aggout,) = _scatter_layer(rowp, colp, g, zeros_in)
        h_new, g_new = _dense_layer(aggout, h, disb)
        return (h_new, g_new), h_new

    _, hs = lax.scan(_body, (xp, g0), None, length=3)
    out = _final_call(coef, xp, hs[0], hs[1], hs[2], W, b.reshape(1, D))
    return out[:N]


# speedup vs baseline: 5.4106x; 1.0000x over previous
import functools

import jax
import jax.numpy as jnp
from jax import lax
from jax.experimental import pallas as pl
from jax.experimental.pallas import tpu as pltpu
from jax.experimental.pallas import tpu_sc as plsc

N = 10000
D = 128
EN = 320000
NC, NS, L = 2, 16, 16
NW = NC * NS
NP = 10240
RPT = NP // NW
NPH = NP + 128
ZR = NPH // NS
HALF = NP // NC
AGGR = HALF + 128
ARPT = AGGR // NS
EPT = 10240
EP = EPT * NW
ECH = 128
EG = 2048
NCG = EPT // EG
CPG = EG // ECH
DCH = 64
NDC = RPT // DCH
EPS = EP // NS
NCGS = EPS // EG

_mesh = plsc.VectorSubcoreMesh(
    core_axis_name="c", subcore_axis_name="s", num_cores=NC, num_subcores=NS)


@functools.partial(
    pl.kernel,
    out_type=[
        jax.ShapeDtypeStruct((NC * NPH, D), jnp.float32),
    ],
    mesh=_mesh,
    scratch_types=[
        pltpu.VMEM((EG,), jnp.int32),
        pltpu.VMEM((ECH,), jnp.int32),
        pltpu.VMEM((ECH, D), jnp.float32),
        pltpu.VMEM_SHARED((NPH, D), jnp.float32),
    ],
)
def _deg_kernel(colp, ones_in, zeros_in, partials, colbuf, idxbuf, ones,
                deg_sh):
    c = lax.axis_index("c")
    s = lax.axis_index("s")
    wid = c * NS + s

    pltpu.sync_copy(ones_in, ones)
    pltpu.sync_copy(zeros_in.at[pl.ds(0, ZR)], deg_sh.at[pl.ds(s * ZR, ZR)])
    plsc.subcore_barrier()

    def group(g, _):
        gb = wid * EPT + g * EG
        pltpu.sync_copy(colp.at[pl.ds(gb, EG)], colbuf)

        def chunk(k, _):
            def cpy(i, _):
                idxbuf[pl.ds(i * L, L)] = colbuf[pl.ds(k * ECH + i * L, L)]
                return 0
            lax.fori_loop(0, ECH // L, cpy, 0)
            pltpu.sync_copy(ones, deg_sh.at[idxbuf], add=True)
            return 0
        lax.fori_loop(0, CPG, chunk, 0)
        return 0
    lax.fori_loop(0, NCG, group, 0)
    plsc.subcore_barrier()

    pltpu.sync_copy(deg_sh.at[pl.ds(s * ZR, ZR)],
                    partials.at[pl.ds(c * NPH + s * ZR, ZR)])


@functools.partial(
    pl.kernel,
    out_type=[
        jax.ShapeDtypeStruct((NP, L), jnp.float32),
        jax.ShapeDtypeStruct((NP, D), jnp.float32),
        jax.ShapeDtypeStruct((L,), jnp.float32),
    ],
    mesh=_mesh,
    scratch_types=[
        pltpu.VMEM((DCH, D), jnp.float32),
        pltpu.VMEM((DCH, D), jnp.float32),
        pltpu.VMEM((DCH, L), jnp.float32),
        pltpu.VMEM((DCH, D), jnp.float32),
        pltpu.VMEM((2, L), jnp.float32),
        pltpu.VMEM((L,), jnp.float32),
    ],
)
def _dis_kernel(partials, xp, kw, disb, g0, coef, pbufa, pbufb, dsb, xbuf,
                kbuf, cbuf):
    c = lax.axis_index("c")
    s = lax.axis_index("s")
    wid = c * NS + s
    base = wid * RPT
    iot = lax.iota(jnp.int32, L)

    def dchunk(ci, _):
        o = base + ci * DCH
        pltpu.sync_copy(partials.at[pl.ds(o, DCH)], pbufa)
        pltpu.sync_copy(partials.at[pl.ds(NPH + o, DCH)], pbufb)
        pltpu.sync_copy(xp.at[pl.ds(o, DCH)], xbuf)

        def rbody(gi, _):
            for i in range(L):
                r = gi * L + i
                deg = pbufa[r, pl.ds(0, L)] + pbufb[r, pl.ds(0, L)]
                deg = jnp.where(deg == 0.0, 1.0, deg)
                y = 1.0 / deg
                for _ in range(24):
                    y = y * (1.5 - 0.5 * deg * y * y)
                dsb[r, :] = y
                for j in range(D // L):
                    sl = pl.ds(j * L, L)
                    xbuf[r, sl] = xbuf[r, sl] * y
            return 0
        lax.fori_loop(0, DCH // L, rbody, 0)
        pltpu.sync_copy(dsb, disb.at[pl.ds(o, DCH)])
        pltpu.sync_copy(xbuf, g0.at[pl.ds(o, DCH)])
        return 0
    lax.fori_loop(0, NDC, dchunk, 0)

    @pl.when(wid == 0)
    def _():
        pltpu.sync_copy(kw, kbuf)
        kv = kbuf[0, :]
        e2k = jnp.exp(2.0 * kv)
        th = 1.0 - 2.0 / (e2k + 1.0)
        csp = 1.0 / (1.0 + jnp.exp(-kbuf[1, :]))
        cv = jnp.where(iot < 3, csp * th,
                       jnp.where(iot == 3, 1.0 - csp, 0.0))
        cbuf[...] = cv
        pltpu.sync_copy(cbuf, coef)


@functools.partial(
    pl.kernel,
    out_type=[
        jax.ShapeDtypeStruct((NC * NP, D), jnp.float32),
    ],
    mesh=_mesh,
    scratch_types=[
        pltpu.VMEM((EG,), jnp.int32),
        pltpu.VMEM((EG,), jnp.int32),
        pltpu.VMEM((ECH,), jnp.int32),
        pltpu.VMEM((ECH,), jnp.int32),
        pltpu.VMEM((ECH,), jnp.int32),
        pltpu.VMEM((ECH,), jnp.int32),
        pltpu.VMEM((ECH, D), jnp.float32),
        pltpu.VMEM((ECH, D), jnp.float32),
        pltpu.VMEM_SHARED((NPH, D), jnp.float32),
        pltpu.SemaphoreType.DMA,
        pltpu.SemaphoreType.DMA,
        pltpu.SemaphoreType.DMA,
        pltpu.SemaphoreType.DMA,
    ],
)
def _scatter_layer(rowp, colp, g_in, zeros_in, aggout, rgbuf, cgbuf,
                   rbuf0, rbuf1, idxbuf0, idxbuf1, gbuf0, gbuf1, agg_sh,
                   sem0, sem1, ssem0, ssem1):
    c = lax.axis_index("c")
    s = lax.axis_index("s")
    wid = c * NS + s
    rbufs = (rbuf0, rbuf1)
    idxbufs = (idxbuf0, idxbuf1)
    gbufs = (gbuf0, gbuf1)
    sems = (sem0, sem1)
    ssems = (ssem0, ssem1)

    pltpu.sync_copy(zeros_in, agg_sh.at[pl.ds(s * ZR, ZR)])
    plsc.subcore_barrier()

    def build_idx(k, rb, ib):
        def cpy(i, _):
            rb[pl.ds(i * L, L)] = rgbuf[pl.ds(k * ECH + i * L, L)]
            cv = cgbuf[pl.ds(k * ECH + i * L, L)]
            ib[pl.ds(i * L, L)] = jnp.minimum(cv, NP)
            return 0
        lax.fori_loop(0, ECH // L, cpy, 0)

    def group(g, _):
        gb = wid * EPT + g * EG
        pltpu.sync_copy(rowp.at[pl.ds(gb, EG)], rgbuf)
        pltpu.sync_copy(colp.at[pl.ds(gb, EG)], cgbuf)

        descs = [None, None]
        sdescs = [None, None]
        for k in range(CPG):
            b = k % 2
            if k >= 2:
                sdescs[b].wait()
            build_idx(k, rbufs[b], idxbufs[b])
            descs[b] = pltpu.async_copy(g_in.at[rbufs[b]], gbufs[b], sems[b])
            if k > 0:
                descs[1 - b].wait()
                sdescs[1 - b] = pltpu.async_copy(
                    gbufs[1 - b], agg_sh.at[idxbufs[1 - b]], ssems[1 - b],
                    add=True)
        descs[1].wait()
        sdescs[1] = pltpu.async_copy(gbufs[1], agg_sh.at[idxbufs[1]],
                                     ssems[1], add=True)
        sdescs[0].wait()
        sdescs[1].wait()
        return 0
    lax.fori_loop(0, NCG, group, 0)
    plsc.subcore_barrier()

    pltpu.sync_copy(agg_sh.at[pl.ds(s * (NP // NS), NP // NS)],
                    aggout.at[pl.ds(c * NP + s * (NP // NS), NP // NS)])


@functools.partial(
    pl.kernel,
    out_type=[
        jax.ShapeDtypeStruct((NP, D), jnp.float32),
        jax.ShapeDtypeStruct((NP, D), jnp.float32),
    ],
    mesh=_mesh,
    scratch_types=[
        pltpu.VMEM((DCH, D), jnp.float32),
        pltpu.VMEM((DCH, D), jnp.float32),
        pltpu.VMEM((DCH, D), jnp.float32),
        pltpu.VMEM((DCH, L), jnp.float32),
    ],
)
def _dense_layer(aggout, h_in, disb, h_out, g_out, bufa, bufb, hbuf, dbuf):
    c = lax.axis_index("c")
    s = lax.axis_index("s")
    wid = c * NS + s
    base = wid * RPT

    def dchunk(ci, _):
        gb = base + ci * DCH
        pltpu.sync_copy(aggout.at[pl.ds(gb, DCH)], bufa)
        pltpu.sync_copy(aggout.at[pl.ds(NP + gb, DCH)], bufb)
        pltpu.sync_copy(h_in.at[pl.ds(gb, DCH)], hbuf)
        pltpu.sync_copy(disb.at[pl.ds(gb, DCH)], dbuf)

        def dense(gi, _):
            for i in range(L):
                r = gi * L + i
                sp = dbuf[r, :]
                for j in range(D // L):
                    sl = pl.ds(j * L, L)
                    hnew = hbuf[r, sl] - sp * (bufa[r, sl] + bufb[r, sl])
                    hbuf[r, sl] = hnew
                    bufa[r, sl] = sp * hnew
            return 0
        lax.fori_loop(0, DCH // L, dense, 0)
        pltpu.sync_copy(hbuf, h_out.at[pl.ds(gb, DCH)])
        pltpu.sync_copy(bufa, g_out.at[pl.ds(gb, DCH)])
        return 0
    lax.fori_loop(0, NDC, dchunk, 0)


_BM = 256


def _final_body(coef_ref, x_ref, h1_ref, h2_ref, h3_ref, w_ref, b_ref, o_ref):
    s0 = coef_ref[0]
    s1 = coef_ref[1]
    s2 = coef_ref[2]
    s3 = coef_ref[3]
    mixed = (s3 * x_ref[...] + s0 * h1_ref[...] + s1 * h2_ref[...]
             + s2 * h3_ref[...])
    acc = lax.dot_general(mixed, w_ref[...], (((1,), (1,)), ((), ())),
                          preferred_element_type=jnp.float32)
    o_ref[...] = jnp.maximum(acc + b_ref[...], 0.0)


_final_call = pl.pallas_call(
    _final_body,
    grid=(NP // _BM,),
    in_specs=[
        pl.BlockSpec(memory_space=pltpu.SMEM),
        pl.BlockSpec((_BM, D), lambda i: (i, 0)),
        pl.BlockSpec((_BM, D), lambda i: (i, 0)),
        pl.BlockSpec((_BM, D), lambda i: (i, 0)),
        pl.BlockSpec((_BM, D), lambda i: (i, 0)),
        pl.BlockSpec((D, D), lambda i: (0, 0)),
        pl.BlockSpec((1, D), lambda i: (0, 0)),
    ],
    out_specs=pl.BlockSpec((_BM, D), lambda i: (i, 0)),
    out_shape=jax.ShapeDtypeStruct((NP, D), jnp.float32),
)


def kernel(x, edge_index, k_values, weighting, W, b):
    x = x.astype(jnp.float32)
    ei = edge_index.astype(jnp.int32)
    row, col = ei[0], ei[1]
    rowp = jnp.concatenate([row, jnp.zeros((EP - EN,), jnp.int32)])
    colp = jnp.concatenate([col, jnp.full((EP - EN,), NP, jnp.int32)])
    xp = jnp.concatenate([x, jnp.zeros((NP - N, D), jnp.float32)])
    kw0 = jnp.concatenate([k_values.reshape(3).astype(jnp.float32),
                           weighting.reshape(1).astype(jnp.float32),
                           jnp.zeros((L - 4,), jnp.float32)])
    kw1 = jnp.broadcast_to(weighting.reshape(1).astype(jnp.float32), (L,))
    kw = jnp.stack([kw0, kw1])
    ones_in = jnp.ones((ECH, D), jnp.float32)
    zeros_in = jnp.zeros((ZR, D), jnp.float32)

    (partials,) = _deg_kernel(colp, ones_in, zeros_in)
    disb, g0, coef = _dis_kernel(partials, xp, kw)

    def _body(carry, _):
        h, g = carry
        (aggout,) = _scatter_layer(rowp, colp, g, zeros_in)
        h_new, g_new = _dense_layer(aggout, h, disb)
        return (h_new, g_new), h_new

    _, hs = lax.scan(_body, (xp, g0), None, length=3)
    out = _final_call(coef, xp, hs[0], hs[1], hs[2], W, b.reshape(1, D))
    return out[:N]
